# baseline (speedup 1.0000x reference)
import jax
import jax.numpy as jnp
from jax.experimental import pallas as pl

_N = 67108864
_COLS = 8192
_ROWS = _N // _COLS
_BLK = 256


def _copy_kernel(x_ref, o_ref):
    o_ref[...] = x_ref[...]

    @pl.when(pl.program_id(0) == 0)
    def _zero_first():
        row = x_ref[0:1, :]
        col = jax.lax.broadcasted_iota(jnp.int32, (1, _COLS), 1)
        o_ref[0:1, :] = jnp.where(col == 0, jnp.float32(0.0), row)


def kernel(x):
    x2 = x.reshape(_ROWS, _COLS)
    out = pl.pallas_call(
        _copy_kernel,
        grid=(_ROWS // _BLK,),
        in_specs=[pl.BlockSpec((_BLK, _COLS), lambda i: (i, 0))],
        out_specs=pl.BlockSpec((_BLK, _COLS), lambda i: (i, 0)),
        out_shape=jax.ShapeDtypeStruct((_ROWS, _COLS), x.dtype),
    )(x2)
    return out.reshape(_N)
